# Initial kernel scaffold; baseline (speedup 1.0000x reference)
import jax
import jax.numpy as jnp
from jax import lax
from jax.experimental import pallas as pl
from jax.experimental.pallas import tpu as pltpu
from jax.experimental.pallas import tpu_sc as plsc

NC = 2
NS = 16
LANES = 16
CHUNK = 128
DGROUPS = 8


def _sc_segment_partials(x, srcs, dsts, ws, n_nodes, k_chunks):
    n_feat = x.shape[1]

    def body(x_hbm, src_hbm, dst_hbm, w_hbm, out_hbm,
             src_v, dst_v, w_v, rows_v, sem, feat_sh):
        cid = lax.axis_index("c")
        sid = lax.axis_index("s")
        wid = sid * NC + cid

        zero = jnp.zeros((LANES,), jnp.float32)

        def zrow(i, carry):
            for g in range(DGROUPS):
                rows_v[i, pl.ds(g * LANES, LANES)] = zero
            return carry

        lax.fori_loop(0, CHUNK, zrow, 0)

        rows_per_tile = n_nodes // NS
        off = sid * rows_per_tile
        done = 0
        while done < rows_per_tile:
            step = min(CHUNK, rows_per_tile - done)
            pltpu.sync_copy(rows_v.at[pl.ds(0, step)],
                            feat_sh.at[pl.ds(off + done, step)])
            done += step
        plsc.subcore_barrier()

        base = wid * k_chunks
        pltpu.sync_copy(src_hbm.at[pl.ds(base, k_chunks)], src_v)
        pltpu.sync_copy(dst_hbm.at[pl.ds(base, k_chunks)], dst_v)
        pltpu.sync_copy(w_hbm.at[pl.ds(base, k_chunks)], w_v)

        def chunk_body(j, carry):
            pltpu.async_copy(x_hbm.at[src_v.at[j]], rows_v, sem).wait()

            def mul_row(i, c):
                w_s = w_v[j, i]
                for g in range(DGROUPS):
                    sl = pl.ds(g * LANES, LANES)
                    rows_v[i, sl] = rows_v[i, sl] * w_s
                return c

            lax.fori_loop(0, CHUNK, mul_row, 0)

            pltpu.sync_copy(rows_v, feat_sh.at[dst_v.at[j]], add=True)
            return carry

        lax.fori_loop(0, k_chunks, chunk_body, 0)
        plsc.subcore_barrier()

        done = 0
        while done < rows_per_tile:
            step = min(CHUNK, rows_per_tile - done)
            sl = pl.ds(off + done, step)
            pltpu.sync_copy(feat_sh.at[sl], rows_v.at[pl.ds(0, step)])
            pltpu.sync_copy(rows_v.at[pl.ds(0, step)], out_hbm.at[cid].at[sl])
            done += step

    mesh = plsc.VectorSubcoreMesh(core_axis_name="c", subcore_axis_name="s")
    fn = pl.kernel(
        body,
        out_type=jax.ShapeDtypeStruct((NC, n_nodes, n_feat), jnp.float32),
        mesh=mesh,
        scratch_types=[
            pltpu.VMEM((k_chunks, CHUNK), jnp.int32),
            pltpu.VMEM((k_chunks, CHUNK), jnp.int32),
            pltpu.VMEM((k_chunks, CHUNK), jnp.float32),
            pltpu.VMEM((CHUNK, n_feat), jnp.float32),
            pltpu.SemaphoreType.DMA,
            pltpu.VMEM_SHARED((n_nodes, n_feat), jnp.float32),
        ],
    )
    return fn(x, srcs, dsts, ws)


def _tc_dense(fp, w, b2, s2, o2):
    n_nodes, n_feat = fp.shape[1], fp.shape[2]

    def body(fp_ref, w_ref, b_ref, s_ref, o_ref, out_ref):
        f = fp_ref[0] + fp_ref[1]
        h = lax.dot_general(f, w_ref[...], (((1,), (1,)), ((), ())),
                            preferred_element_type=jnp.float32)
        h = h + b_ref[...]
        h = jnp.where(h > 0, h, jnp.expm1(h))
        mean = jnp.mean(h, axis=1, keepdims=True)
        c = h - mean
        var = jnp.mean(c * c, axis=1, keepdims=True) + 1e-9
        out_ref[...] = c * s_ref[...] * lax.rsqrt(var) + o_ref[...]

    return pl.pallas_call(
        body,
        out_shape=jax.ShapeDtypeStruct((n_nodes, n_feat), jnp.float32),
    )(fp, w, b2, s2, o2)


def kernel(x, edge_index, edge_weight, W, b, scale, offset,
           sampled_nodes, nodes_per_layer, iterations, epoch):
    n_nodes = x.shape[0]
    n_edges = edge_weight.shape[0]
    nw = NC * NS

    k_chunks = -(-n_edges // (nw * CHUNK))
    e_pad = nw * k_chunks * CHUNK
    pad = e_pad - n_edges

    src = jnp.pad(edge_index[0], (0, pad)).reshape(nw * k_chunks, CHUNK)
    dst = jnp.pad(edge_index[1], (0, pad)).reshape(nw * k_chunks, CHUNK)
    ws = jnp.pad(edge_weight, (0, pad)).reshape(nw * k_chunks, CHUNK)

    fp = _sc_segment_partials(x, src, dst, ws, n_nodes, k_chunks)
    return _tc_dense(fp, W,
                     b.reshape(1, -1), scale.reshape(1, -1),
                     offset.reshape(1, -1))


# speedup vs baseline: 3.0624x; 3.0624x over previous
import jax
import jax.numpy as jnp
from jax import lax
from jax.experimental import pallas as pl
from jax.experimental.pallas import tpu as pltpu
from jax.experimental.pallas import tpu_sc as plsc

NC = 2
NS = 16
LANES = 16
CHUNK = 128
DGROUPS = 8


def _sc_segment_partials(x, srcs, dsts, ws, n_nodes, n_pad, k_chunks):
    n_feat = x.shape[1]
    rows_per_tile = n_pad // NS

    def body(x_hbm, src_hbm, dst_hbm, w_hbm, out_hbm,
             src_v, dst_v, w_v, rows_v, sem, feat_sh):
        cid = lax.axis_index("c")
        sid = lax.axis_index("s")
        wid = sid * NC + cid

        zero = jnp.zeros((LANES,), jnp.float32)

        def zrow(i, carry):
            for g in range(DGROUPS):
                rows_v[i, pl.ds(g * LANES, LANES)] = zero
            return carry

        lax.fori_loop(0, CHUNK, zrow, 0)

        off = sid * rows_per_tile
        done = 0
        while done < rows_per_tile:
            step = min(CHUNK, rows_per_tile - done)
            pltpu.sync_copy(rows_v.at[pl.ds(0, step)],
                            feat_sh.at[pl.ds(off + done, step)])
            done += step
        plsc.subcore_barrier()

        base = wid * k_chunks
        pltpu.sync_copy(src_hbm.at[pl.ds(base, k_chunks)], src_v)
        pltpu.sync_copy(dst_hbm.at[pl.ds(base, k_chunks)], dst_v)
        pltpu.sync_copy(w_hbm.at[pl.ds(base, k_chunks)], w_v)

        def chunk_body(j, carry):
            pltpu.async_copy(x_hbm.at[src_v.at[j]], rows_v, sem).wait()

            def mul_block(bi, c):
                wv = w_v[j, pl.ds(bi * LANES, LANES)]
                for l in range(LANES):
                    w_s = wv[l]
                    row = bi * LANES + l
                    for g in range(DGROUPS):
                        sl = pl.ds(g * LANES, LANES)
                        rows_v[row, sl] = rows_v[row, sl] * w_s
                return c

            lax.fori_loop(0, CHUNK // LANES, mul_block, 0)

            pltpu.sync_copy(rows_v, feat_sh.at[dst_v.at[j]], add=True)
            return carry

        lax.fori_loop(0, k_chunks, chunk_body, 0)
        plsc.subcore_barrier()

        done = 0
        while done < rows_per_tile:
            step = min(CHUNK, rows_per_tile - done)
            sl = pl.ds(off + done, step)
            pltpu.sync_copy(feat_sh.at[sl], rows_v.at[pl.ds(0, step)])
            pltpu.sync_copy(rows_v.at[pl.ds(0, step)], out_hbm.at[cid].at[sl])
            done += step

    mesh = plsc.VectorSubcoreMesh(core_axis_name="c", subcore_axis_name="s")
    fn = pl.kernel(
        body,
        out_type=jax.ShapeDtypeStruct((NC, n_pad, n_feat), jnp.float32),
        mesh=mesh,
        scratch_types=[
            pltpu.VMEM((k_chunks, CHUNK), jnp.int32),
            pltpu.VMEM((k_chunks, CHUNK), jnp.int32),
            pltpu.VMEM((k_chunks, CHUNK), jnp.float32),
            pltpu.VMEM((CHUNK, n_feat), jnp.float32),
            pltpu.SemaphoreType.DMA,
            pltpu.VMEM_SHARED((n_pad, n_feat), jnp.float32),
        ],
    )
    return fn(x, srcs, dsts, ws)


def _tc_dense(fp, w, b2, s2, o2, n_nodes):
    n_feat = fp.shape[2]

    def body(fp_ref, w_ref, b_ref, s_ref, o_ref, out_ref):
        f = fp_ref[0, :n_nodes, :] + fp_ref[1, :n_nodes, :]
        h = lax.dot_general(f, w_ref[...], (((1,), (1,)), ((), ())),
                            preferred_element_type=jnp.float32)
        h = h + b_ref[...]
        h = jnp.where(h > 0, h, jnp.exp(h) - 1.0)
        mean = jnp.mean(h, axis=1, keepdims=True)
        c = h - mean
        var = jnp.mean(c * c, axis=1, keepdims=True) + 1e-9
        out_ref[...] = c * s_ref[...] * lax.rsqrt(var) + o_ref[...]

    return pl.pallas_call(
        body,
        out_shape=jax.ShapeDtypeStruct((n_nodes, n_feat), jnp.float32),
    )(fp, w, b2, s2, o2)


def kernel(x, edge_index, edge_weight, W, b, scale, offset,
           sampled_nodes, nodes_per_layer, iterations, epoch):
    n_nodes = x.shape[0]
    n_edges = edge_weight.shape[0]
    nw = NC * NS

    k_chunks = -(-(-(-n_edges // (nw * CHUNK))) // 8) * 8
    e_pad = nw * k_chunks * CHUNK
    pad = e_pad - n_edges

    src = jnp.pad(edge_index[0], (0, pad)).reshape(nw * k_chunks, CHUNK)
    dst = jnp.pad(edge_index[1], (0, pad)).reshape(nw * k_chunks, CHUNK)
    ws = jnp.pad(edge_weight, (0, pad)).reshape(nw * k_chunks, CHUNK)

    n_pad = -(-n_nodes // (NS * 8)) * NS * 8
    fp = _sc_segment_partials(x, src, dst, ws, n_nodes, n_pad, k_chunks)
    return _tc_dense(fp, W,
                     b.reshape(1, -1), scale.reshape(1, -1),
                     offset.reshape(1, -1), n_nodes)
